# Initial kernel scaffold; baseline (speedup 1.0000x reference)
import jax
import jax.numpy as jnp
from jax.experimental import pallas as pl


def kernel(x, edge_index, batch, lin_w, lin_b, ggc0_w, ggc0_wih, ggc0_whh, ggc0_bih, ggc0_bhh, pool0_w, ggc1_w, ggc1_wih, ggc1_whh, ggc1_bih, ggc1_bhh, pool1_w, gate_w, gate_b):
    raise NotImplementedError("write your pallas kernel here")


# speedup vs baseline: 3.1038x; 3.1038x over previous
import functools

import jax
import jax.numpy as jnp
from jax import lax
from jax.experimental import pallas as pl
from jax.experimental.pallas import tpu as pltpu
from jax.experimental.pallas import tpu_sc as plsc

N = 10000
E = 320000
C = 256
G = 16
NGRU = 3
BN = 1000
CH = 1000
NS = 16
K = 80
NCHUNK = E // NS // K
NPAD = 10240
RPS = NPAD // NS


def _edge_agg(m0, m1, srcr4, dstr, zrows):
    mcat = jnp.concatenate([m0[:, :64], m0[:, 64:], m1[:, :64], m1[:, 64:]],
                           axis=0)
    mesh = plsc.VectorSubcoreMesh(core_axis_name="c", subcore_axis_name="s")

    @functools.partial(
        pl.kernel,
        mesh=mesh,
        out_type=jax.ShapeDtypeStruct((4, NPAD, 64), jnp.float32),
        scratch_types=[
            pltpu.VMEM((NCHUNK, K), jnp.int32),
            pltpu.VMEM((NCHUNK, K), jnp.int32),
            pltpu.VMEM((K, 64), jnp.float32),
            pltpu.VMEM_SHARED((NPAD, 64), jnp.float32),
            pltpu.SemaphoreType.DMA,
        ],
        compiler_params=pltpu.CompilerParams(use_tc_tiling_on_sc=False),
    )
    def k(m_hbm, src_hbm, dst_hbm, z_hbm, o_hbm,
          src_v, dst_v, rows_v, agg_sh, sem):
        c = lax.axis_index("c")
        s = lax.axis_index("s")
        pltpu.sync_copy(dst_hbm.at[s], dst_v)
        for p in range(2):
            q = 2 * c + p
            pltpu.sync_copy(z_hbm, agg_sh.at[pl.ds(s * RPS, RPS)])
            pltpu.sync_copy(src_hbm.at[q].at[s], src_v)
            plsc.subcore_barrier()

            def chunk(j, carry):
                pltpu.async_copy(m_hbm.at[src_v.at[j]], rows_v, sem).wait()
                pltpu.sync_copy(rows_v, agg_sh.at[dst_v.at[j]], add=True)
                return carry

            lax.fori_loop(0, NCHUNK, chunk, 0)
            plsc.subcore_barrier()
            pltpu.sync_copy(agg_sh.at[pl.ds(s * RPS, RPS)],
                            o_hbm.at[q].at[pl.ds(s * RPS, RPS)])
            plsc.subcore_barrier()

    op = k(mcat, srcr4, dstr, zrows)
    return (jnp.concatenate([op[0, :N], op[1, :N]], axis=1),
            jnp.concatenate([op[2, :N], op[3, :N]], axis=1))


def _lin_body(x_ref, w_ref, b_ref, o_ref):
    o_ref[...] = jax.nn.relu(
        jnp.dot(x_ref[...], w_ref[...], preferred_element_type=jnp.float32)
        + b_ref[...])


def _lin(x, w, b2):
    F = x.shape[1]
    return pl.pallas_call(
        _lin_body,
        grid=(N // BN,),
        in_specs=[pl.BlockSpec((BN, F), lambda i: (i, 0)),
                  pl.BlockSpec((F, C), lambda i: (0, 0)),
                  pl.BlockSpec((1, C), lambda i: (0, 0))],
        out_specs=pl.BlockSpec((BN, C), lambda i: (i, 0)),
        out_shape=jax.ShapeDtypeStruct((N, C), jnp.float32),
    )(x, w, b2)


def _m_body(h_ref, k_ref, w_ref, m0_ref, m1_ref):
    hm = h_ref[...] * k_ref[...]
    m = jnp.dot(hm, w_ref[...], preferred_element_type=jnp.float32)
    m0_ref[...] = m[:, :128]
    m1_ref[...] = m[:, 128:]


def _m_step(h, keep_col, w):
    return pl.pallas_call(
        _m_body,
        grid=(N // BN,),
        in_specs=[pl.BlockSpec((BN, C), lambda i: (i, 0)),
                  pl.BlockSpec((BN, 1), lambda i: (i, 0)),
                  pl.BlockSpec((C, C), lambda i: (0, 0))],
        out_specs=[pl.BlockSpec((BN, 128), lambda i: (i, 0)),
                   pl.BlockSpec((BN, 128), lambda i: (i, 0))],
        out_shape=[jax.ShapeDtypeStruct((N, 128), jnp.float32),
                   jax.ShapeDtypeStruct((N, 128), jnp.float32)],
    )(h, keep_col, w)


def _gru_body(fuse_m, final_relu, a0_ref, a1_ref, h_ref, wih_ref, whh_ref,
              bih_ref, bhh_ref, k_ref, wn_ref, hn_ref, *m_refs):
    agg = jnp.concatenate([a0_ref[...], a1_ref[...]], axis=1)
    gi = jnp.dot(agg, wih_ref[...], preferred_element_type=jnp.float32) + bih_ref[...]
    gh = jnp.dot(h_ref[...], whh_ref[...], preferred_element_type=jnp.float32) + bhh_ref[...]
    r = jax.nn.sigmoid(gi[:, :C] + gh[:, :C])
    z = jax.nn.sigmoid(gi[:, C:2 * C] + gh[:, C:2 * C])
    n = jnp.tanh(gi[:, 2 * C:] + r * gh[:, 2 * C:])
    hn = (1.0 - z) * n + z * h_ref[...]
    if final_relu:
        hn = jax.nn.relu(hn)
    hn_ref[...] = hn
    if fuse_m:
        m = jnp.dot(hn * k_ref[...], wn_ref[...],
                    preferred_element_type=jnp.float32)
        m_refs[0][...] = m[:, :128]
        m_refs[1][...] = m[:, 128:]


def _gru(a0, a1, h, wih_t, whh_t, bih2, bhh2, keep_col, w_next, final_relu):
    fuse_m = w_next is not None
    out_shape = [jax.ShapeDtypeStruct((N, C), jnp.float32)]
    out_specs = [pl.BlockSpec((BN, C), lambda i: (i, 0))]
    if fuse_m:
        out_shape += [jax.ShapeDtypeStruct((N, 128), jnp.float32)] * 2
        out_specs += [pl.BlockSpec((BN, 128), lambda i: (i, 0))] * 2
        wn = w_next
    else:
        wn = jnp.zeros((C, C), jnp.float32)
    res = pl.pallas_call(
        functools.partial(_gru_body, fuse_m, final_relu),
        grid=(N // BN,),
        in_specs=[pl.BlockSpec((BN, 128), lambda i: (i, 0)),
                  pl.BlockSpec((BN, 128), lambda i: (i, 0)),
                  pl.BlockSpec((BN, C), lambda i: (i, 0)),
                  pl.BlockSpec((C, 3 * C), lambda i: (0, 0)),
                  pl.BlockSpec((C, 3 * C), lambda i: (0, 0)),
                  pl.BlockSpec((1, 3 * C), lambda i: (0, 0)),
                  pl.BlockSpec((1, 3 * C), lambda i: (0, 0)),
                  pl.BlockSpec((BN, 1), lambda i: (i, 0)),
                  pl.BlockSpec((C, C), lambda i: (0, 0))],
        out_specs=out_specs,
        out_shape=out_shape,
    )(a0, a1, h, wih_t, whh_t, bih2, bhh2, keep_col, wn)
    return res if fuse_m else res[0]


def _score_body(h_ref, k_ref, b_ref, pw_ref, s_ref, w_ref):
    pw = pw_ref[...]
    nrm = jnp.sqrt(jnp.sum(pw * pw))
    sc = jnp.tanh(jnp.sum(h_ref[...] * pw, axis=1, keepdims=True) / nrm)
    s_ref[...] = sc
    sm = jnp.where(k_ref[...] > 0, sc, -2.0)
    w_ref[...] = 8.0 * b_ref[...] - sm


def _topk_score(h, keep_col, batch_col, pw2):
    return pl.pallas_call(
        _score_body,
        grid=(N // BN,),
        in_specs=[pl.BlockSpec((BN, C), lambda i: (i, 0)),
                  pl.BlockSpec((BN, 1), lambda i: (i, 0)),
                  pl.BlockSpec((BN, 1), lambda i: (i, 0)),
                  pl.BlockSpec((1, C), lambda i: (0, 0))],
        out_specs=[pl.BlockSpec((BN, 1), lambda i: (i, 0)),
                   pl.BlockSpec((BN, 1), lambda i: (i, 0))],
        out_shape=[jax.ShapeDtypeStruct((N, 1), jnp.float32),
                   jax.ShapeDtypeStruct((N, 1), jnp.float32)],
    )(h, keep_col, batch_col, pw2)


def _rank_body(h_ref, s_ref, wc_ref, bc_ref, wr_ref, br_ref, kr_ref,
               hn_ref, kn_ref):
    wi = wc_ref[...]
    bi = bc_ref[...]
    ii = (lax.broadcasted_iota(jnp.int32, (BN, 1), 0).astype(jnp.float32)
          + jnp.float32(BN) * pl.program_id(0))
    lt = jnp.zeros((BN, 1), jnp.float32)
    tie = jnp.zeros((BN, 1), jnp.float32)
    blt = jnp.zeros((BN, 1), jnp.float32)
    kc = jnp.zeros((BN, 1), jnp.float32)
    for cidx in range(N // CH):
        wj = wr_ref[:, pl.ds(cidx * CH, CH)]
        bj = br_ref[:, pl.ds(cidx * CH, CH)]
        kj = kr_ref[:, pl.ds(cidx * CH, CH)]
        jj = (lax.broadcasted_iota(jnp.int32, (1, CH), 1).astype(jnp.float32)
              + jnp.float32(cidx * CH))
        lt += jnp.sum((wj < wi).astype(jnp.float32), axis=1, keepdims=True)
        tie += jnp.sum(((wj == wi) & (jj < ii)).astype(jnp.float32),
                       axis=1, keepdims=True)
        blt += jnp.sum((bj < bi).astype(jnp.float32), axis=1, keepdims=True)
        kc += jnp.sum((bj == bi).astype(jnp.float32) * kj, axis=1, keepdims=True)
    rank = lt + tie - blt
    kk = jnp.ceil(0.5 * kc)
    kn = (rank < kk).astype(jnp.float32)
    kn_ref[...] = kn
    hn_ref[...] = h_ref[...] * s_ref[...] * kn


def _topk_rank(h, score_col, wkey_col, batch_col, wkey_row, batch_row, keep_row):
    return pl.pallas_call(
        _rank_body,
        grid=(N // BN,),
        in_specs=[pl.BlockSpec((BN, C), lambda i: (i, 0)),
                  pl.BlockSpec((BN, 1), lambda i: (i, 0)),
                  pl.BlockSpec((BN, 1), lambda i: (i, 0)),
                  pl.BlockSpec((BN, 1), lambda i: (i, 0)),
                  pl.BlockSpec((1, N), lambda i: (0, 0)),
                  pl.BlockSpec((1, N), lambda i: (0, 0)),
                  pl.BlockSpec((1, N), lambda i: (0, 0))],
        out_specs=[pl.BlockSpec((BN, C), lambda i: (i, 0)),
                   pl.BlockSpec((BN, 1), lambda i: (i, 0))],
        out_shape=[jax.ShapeDtypeStruct((N, C), jnp.float32),
                   jax.ShapeDtypeStruct((N, 1), jnp.float32)],
    )(h, score_col, wkey_col, batch_col, wkey_row, batch_row, keep_row)


def _attpool_body(h_ref, k_ref, b_ref, gw_ref, gb_ref, acc_ref, o_ref):
    h = h_ref[...]
    gate = jnp.dot(h, gw_ref[...], preferred_element_type=jnp.float32) + gb_ref[...]
    keep = k_ref[...]
    gate = jnp.where(keep > 0, gate, -1e9)
    onehot = b_ref[...] == lax.broadcasted_iota(jnp.int32, (1, G), 1).astype(jnp.float32)
    masked = jnp.where(onehot, gate, -1e9)
    gmax = jnp.max(masked, axis=0, keepdims=True)
    e = jnp.where(onehot, jnp.exp(gate - gmax), 0.0) * keep
    den = jnp.sum(e, axis=0, keepdims=True)
    wts = e / (den + 1e-12)
    o_ref[...] = acc_ref[...] + lax.dot_general(
        wts, h, (((0,), (0,)), ((), ())), preferred_element_type=jnp.float32)


def _attpool(h, keep_col, batch_col, gw, gb2, acc):
    return pl.pallas_call(
        _attpool_body,
        in_specs=[pl.BlockSpec((N, C), lambda: (0, 0)),
                  pl.BlockSpec((N, 1), lambda: (0, 0)),
                  pl.BlockSpec((N, 1), lambda: (0, 0)),
                  pl.BlockSpec((C, 1), lambda: (0, 0)),
                  pl.BlockSpec((1, 1), lambda: (0, 0)),
                  pl.BlockSpec((G, C), lambda: (0, 0))],
        out_specs=pl.BlockSpec((G, C), lambda: (0, 0)),
        out_shape=jax.ShapeDtypeStruct((G, C), jnp.float32),
    )(h, keep_col, batch_col, gw, gb2, acc)


def _layer(h, keep_col, srcr4, dstr, zrows, w3, wih_t, whh_t, bih2, bhh2):
    m0, m1 = _m_step(h, keep_col, w3[0])
    for l in range(NGRU):
        a0, a1 = _edge_agg(m0, m1, srcr4, dstr, zrows)
        if l < NGRU - 1:
            h, m0, m1 = _gru(a0, a1, h, wih_t, whh_t, bih2, bhh2,
                             keep_col, w3[l + 1], final_relu=False)
        else:
            h = _gru(a0, a1, h, wih_t, whh_t, bih2, bhh2,
                     keep_col, None, final_relu=True)
    return h


def _topk(h, keep_col, batch_col, batch_row, keep_row, pw):
    score, wkey = _topk_score(h, keep_col, batch_col, pw.reshape(1, C))
    h, kn_col = _topk_rank(h, score, wkey, batch_col,
                           wkey.reshape(1, N), batch_row, keep_row)
    return h, kn_col


def kernel(x, edge_index, batch, lin_w, lin_b, ggc0_w, ggc0_wih, ggc0_whh,
           ggc0_bih, ggc0_bhh, pool0_w, ggc1_w, ggc1_wih, ggc1_whh,
           ggc1_bih, ggc1_bhh, pool1_w, gate_w, gate_b):
    src = edge_index[0].astype(jnp.int32)
    dst = edge_index[1].astype(jnp.int32)
    srcr = src.reshape(NS, NCHUNK, K)
    srcr4 = srcr[None] + (jnp.arange(4, dtype=jnp.int32) * N)[:, None, None, None]
    dstr = dst.reshape(NS, NCHUNK, K)
    zrows = jnp.zeros((RPS, 64), jnp.float32)
    bcol = batch.astype(jnp.float32).reshape(N, 1)
    brow = batch.astype(jnp.float32).reshape(1, N)
    ones_col = jnp.ones((N, 1), jnp.float32)
    ones_row = jnp.ones((1, N), jnp.float32)

    h = _lin(x, lin_w, lin_b.reshape(1, C))
    h = _layer(h, ones_col, srcr4, dstr, zrows, ggc0_w,
               ggc0_wih.T, ggc0_whh.T,
               ggc0_bih.reshape(1, 3 * C), ggc0_bhh.reshape(1, 3 * C))
    h, keep_col = _topk(h, ones_col, bcol, brow, ones_row, pool0_w)
    out = _attpool(h, keep_col, bcol, gate_w, gate_b.reshape(1, 1),
                   jnp.zeros((G, C), jnp.float32))
    h = _layer(h, keep_col, srcr4, dstr, zrows, ggc1_w,
               ggc1_wih.T, ggc1_whh.T,
               ggc1_bih.reshape(1, 3 * C), ggc1_bhh.reshape(1, 3 * C))
    h, keep_col = _topk(h, keep_col, bcol, brow,
                        keep_col.reshape(1, N), pool1_w)
    out = _attpool(h, keep_col, bcol, gate_w, gate_b.reshape(1, 1), out)
    return out


# speedup vs baseline: 4.3196x; 1.3917x over previous
import functools

import jax
import jax.numpy as jnp
from jax import lax
from jax.experimental import pallas as pl
from jax.experimental.pallas import tpu as pltpu
from jax.experimental.pallas import tpu_sc as plsc

N = 10000
E = 320000
C = 256
G = 16
NGRU = 3
BN = 1000
CH = 1000
NS = 16
K = 80
NCHUNK = E // NS // K
NPAD = 10240
RPS = NPAD // NS


def _edge_agg(m0, m1, srcr4, dstr, zrows):
    mcat = jnp.concatenate([m0[:, :64], m0[:, 64:], m1[:, :64], m1[:, 64:]],
                           axis=0)
    mesh = plsc.VectorSubcoreMesh(core_axis_name="c", subcore_axis_name="s")

    @functools.partial(
        pl.kernel,
        mesh=mesh,
        out_type=jax.ShapeDtypeStruct((4, NPAD, 64), jnp.float32),
        scratch_types=[
            pltpu.VMEM((NCHUNK, K), jnp.int32),
            pltpu.VMEM((NCHUNK, K), jnp.int32),
            pltpu.VMEM((K, 64), jnp.float32),
            pltpu.VMEM((K, 64), jnp.float32),
            pltpu.VMEM_SHARED((NPAD, 64), jnp.float32),
            pltpu.SemaphoreType.DMA,
            pltpu.SemaphoreType.DMA,
        ],
        compiler_params=pltpu.CompilerParams(use_tc_tiling_on_sc=False),
    )
    def k(m_hbm, src_hbm, dst_hbm, z_hbm, o_hbm,
          src_v, dst_v, buf0, buf1, agg_sh, sem0, sem1):
        c = lax.axis_index("c")
        s = lax.axis_index("s")
        pltpu.sync_copy(dst_hbm.at[s], dst_v)
        npair = NCHUNK // 2
        for p in range(2):
            q = 2 * c + p
            pltpu.sync_copy(z_hbm, agg_sh.at[pl.ds(s * RPS, RPS)])
            pltpu.sync_copy(src_hbm.at[q].at[s], src_v)
            plsc.subcore_barrier()
            pltpu.async_copy(m_hbm.at[src_v.at[0]], buf0, sem0)

            def pair(i, carry):
                j0 = 2 * i
                pltpu.async_copy(m_hbm.at[src_v.at[j0 + 1]], buf1, sem1)
                pltpu.make_async_copy(m_hbm.at[src_v.at[j0]], buf0,
                                      sem0).wait()
                pltpu.sync_copy(buf0, agg_sh.at[dst_v.at[j0]], add=True)

                @pl.when(i < npair - 1)
                def _():
                    pltpu.async_copy(m_hbm.at[src_v.at[j0 + 2]], buf0, sem0)

                pltpu.make_async_copy(m_hbm.at[src_v.at[j0 + 1]], buf1,
                                      sem1).wait()
                pltpu.sync_copy(buf1, agg_sh.at[dst_v.at[j0 + 1]], add=True)
                return carry

            lax.fori_loop(0, npair, pair, 0)
            plsc.subcore_barrier()
            pltpu.sync_copy(agg_sh.at[pl.ds(s * RPS, RPS)],
                            o_hbm.at[q].at[pl.ds(s * RPS, RPS)])
            plsc.subcore_barrier()

    op = k(mcat, srcr4, dstr, zrows)
    return (jnp.concatenate([op[0, :N], op[1, :N]], axis=1),
            jnp.concatenate([op[2, :N], op[3, :N]], axis=1))


def _lin_body(x_ref, w_ref, b_ref, o_ref):
    o_ref[...] = jax.nn.relu(
        jnp.dot(x_ref[...], w_ref[...], preferred_element_type=jnp.float32)
        + b_ref[...])


def _lin(x, w, b2):
    F = x.shape[1]
    return pl.pallas_call(
        _lin_body,
        grid=(N // BN,),
        in_specs=[pl.BlockSpec((BN, F), lambda i: (i, 0)),
                  pl.BlockSpec((F, C), lambda i: (0, 0)),
                  pl.BlockSpec((1, C), lambda i: (0, 0))],
        out_specs=pl.BlockSpec((BN, C), lambda i: (i, 0)),
        out_shape=jax.ShapeDtypeStruct((N, C), jnp.float32),
    )(x, w, b2)


def _m_body(h_ref, k_ref, w_ref, m0_ref, m1_ref):
    hm = h_ref[...] * k_ref[...]
    m = jnp.dot(hm, w_ref[...], preferred_element_type=jnp.float32)
    m0_ref[...] = m[:, :128]
    m1_ref[...] = m[:, 128:]


def _m_step(h, keep_col, w):
    return pl.pallas_call(
        _m_body,
        grid=(N // BN,),
        in_specs=[pl.BlockSpec((BN, C), lambda i: (i, 0)),
                  pl.BlockSpec((BN, 1), lambda i: (i, 0)),
                  pl.BlockSpec((C, C), lambda i: (0, 0))],
        out_specs=[pl.BlockSpec((BN, 128), lambda i: (i, 0)),
                   pl.BlockSpec((BN, 128), lambda i: (i, 0))],
        out_shape=[jax.ShapeDtypeStruct((N, 128), jnp.float32),
                   jax.ShapeDtypeStruct((N, 128), jnp.float32)],
    )(h, keep_col, w)


def _gru_body(fuse_m, final_relu, a0_ref, a1_ref, h_ref, wih_ref, whh_ref,
              bih_ref, bhh_ref, k_ref, wn_ref, hn_ref, *m_refs):
    agg = jnp.concatenate([a0_ref[...], a1_ref[...]], axis=1)
    gi = jnp.dot(agg, wih_ref[...], preferred_element_type=jnp.float32) + bih_ref[...]
    gh = jnp.dot(h_ref[...], whh_ref[...], preferred_element_type=jnp.float32) + bhh_ref[...]
    r = jax.nn.sigmoid(gi[:, :C] + gh[:, :C])
    z = jax.nn.sigmoid(gi[:, C:2 * C] + gh[:, C:2 * C])
    n = jnp.tanh(gi[:, 2 * C:] + r * gh[:, 2 * C:])
    hn = (1.0 - z) * n + z * h_ref[...]
    if final_relu:
        hn = jax.nn.relu(hn)
    hn_ref[...] = hn
    if fuse_m:
        m = jnp.dot(hn * k_ref[...], wn_ref[...],
                    preferred_element_type=jnp.float32)
        m_refs[0][...] = m[:, :128]
        m_refs[1][...] = m[:, 128:]


def _gru(a0, a1, h, wih_t, whh_t, bih2, bhh2, keep_col, w_next, final_relu):
    fuse_m = w_next is not None
    out_shape = [jax.ShapeDtypeStruct((N, C), jnp.float32)]
    out_specs = [pl.BlockSpec((BN, C), lambda i: (i, 0))]
    if fuse_m:
        out_shape += [jax.ShapeDtypeStruct((N, 128), jnp.float32)] * 2
        out_specs += [pl.BlockSpec((BN, 128), lambda i: (i, 0))] * 2
        wn = w_next
    else:
        wn = jnp.zeros((C, C), jnp.float32)
    res = pl.pallas_call(
        functools.partial(_gru_body, fuse_m, final_relu),
        grid=(N // BN,),
        in_specs=[pl.BlockSpec((BN, 128), lambda i: (i, 0)),
                  pl.BlockSpec((BN, 128), lambda i: (i, 0)),
                  pl.BlockSpec((BN, C), lambda i: (i, 0)),
                  pl.BlockSpec((C, 3 * C), lambda i: (0, 0)),
                  pl.BlockSpec((C, 3 * C), lambda i: (0, 0)),
                  pl.BlockSpec((1, 3 * C), lambda i: (0, 0)),
                  pl.BlockSpec((1, 3 * C), lambda i: (0, 0)),
                  pl.BlockSpec((BN, 1), lambda i: (i, 0)),
                  pl.BlockSpec((C, C), lambda i: (0, 0))],
        out_specs=out_specs,
        out_shape=out_shape,
    )(a0, a1, h, wih_t, whh_t, bih2, bhh2, keep_col, wn)
    return res if fuse_m else res[0]


def _score_body(h_ref, k_ref, b_ref, pw_ref, s_ref, w_ref):
    pw = pw_ref[...]
    nrm = jnp.sqrt(jnp.sum(pw * pw))
    sc = jnp.tanh(jnp.sum(h_ref[...] * pw, axis=1, keepdims=True) / nrm)
    s_ref[...] = sc
    sm = jnp.where(k_ref[...] > 0, sc, -2.0)
    w_ref[...] = 8.0 * b_ref[...] - sm


def _topk_score(h, keep_col, batch_col, pw2):
    return pl.pallas_call(
        _score_body,
        grid=(N // BN,),
        in_specs=[pl.BlockSpec((BN, C), lambda i: (i, 0)),
                  pl.BlockSpec((BN, 1), lambda i: (i, 0)),
                  pl.BlockSpec((BN, 1), lambda i: (i, 0)),
                  pl.BlockSpec((1, C), lambda i: (0, 0))],
        out_specs=[pl.BlockSpec((BN, 1), lambda i: (i, 0)),
                   pl.BlockSpec((BN, 1), lambda i: (i, 0))],
        out_shape=[jax.ShapeDtypeStruct((N, 1), jnp.float32),
                   jax.ShapeDtypeStruct((N, 1), jnp.float32)],
    )(h, keep_col, batch_col, pw2)


def _rank_body(h_ref, s_ref, wc_ref, bc_ref, wr_ref, br_ref, kr_ref,
               hn_ref, kn_ref):
    wi = wc_ref[...]
    bi = bc_ref[...]
    ii = (lax.broadcasted_iota(jnp.int32, (BN, 1), 0).astype(jnp.float32)
          + jnp.float32(BN) * pl.program_id(0))
    lt = jnp.zeros((BN, 1), jnp.float32)
    tie = jnp.zeros((BN, 1), jnp.float32)
    blt = jnp.zeros((BN, 1), jnp.float32)
    kc = jnp.zeros((BN, 1), jnp.float32)
    for cidx in range(N // CH):
        wj = wr_ref[:, pl.ds(cidx * CH, CH)]
        bj = br_ref[:, pl.ds(cidx * CH, CH)]
        kj = kr_ref[:, pl.ds(cidx * CH, CH)]
        jj = (lax.broadcasted_iota(jnp.int32, (1, CH), 1).astype(jnp.float32)
              + jnp.float32(cidx * CH))
        lt += jnp.sum((wj < wi).astype(jnp.float32), axis=1, keepdims=True)
        tie += jnp.sum(((wj == wi) & (jj < ii)).astype(jnp.float32),
                       axis=1, keepdims=True)
        blt += jnp.sum((bj < bi).astype(jnp.float32), axis=1, keepdims=True)
        kc += jnp.sum((bj == bi).astype(jnp.float32) * kj, axis=1, keepdims=True)
    rank = lt + tie - blt
    kk = jnp.ceil(0.5 * kc)
    kn = (rank < kk).astype(jnp.float32)
    kn_ref[...] = kn
    hn_ref[...] = h_ref[...] * s_ref[...] * kn


def _topk_rank(h, score_col, wkey_col, batch_col, wkey_row, batch_row, keep_row):
    return pl.pallas_call(
        _rank_body,
        grid=(N // BN,),
        in_specs=[pl.BlockSpec((BN, C), lambda i: (i, 0)),
                  pl.BlockSpec((BN, 1), lambda i: (i, 0)),
                  pl.BlockSpec((BN, 1), lambda i: (i, 0)),
                  pl.BlockSpec((BN, 1), lambda i: (i, 0)),
                  pl.BlockSpec((1, N), lambda i: (0, 0)),
                  pl.BlockSpec((1, N), lambda i: (0, 0)),
                  pl.BlockSpec((1, N), lambda i: (0, 0))],
        out_specs=[pl.BlockSpec((BN, C), lambda i: (i, 0)),
                   pl.BlockSpec((BN, 1), lambda i: (i, 0))],
        out_shape=[jax.ShapeDtypeStruct((N, C), jnp.float32),
                   jax.ShapeDtypeStruct((N, 1), jnp.float32)],
    )(h, score_col, wkey_col, batch_col, wkey_row, batch_row, keep_row)


def _attpool_body(h_ref, k_ref, b_ref, gw_ref, gb_ref, acc_ref, o_ref):
    h = h_ref[...]
    gate = jnp.dot(h, gw_ref[...], preferred_element_type=jnp.float32) + gb_ref[...]
    keep = k_ref[...]
    gate = jnp.where(keep > 0, gate, -1e9)
    onehot = b_ref[...] == lax.broadcasted_iota(jnp.int32, (1, G), 1).astype(jnp.float32)
    masked = jnp.where(onehot, gate, -1e9)
    gmax = jnp.max(masked, axis=0, keepdims=True)
    e = jnp.where(onehot, jnp.exp(gate - gmax), 0.0) * keep
    den = jnp.sum(e, axis=0, keepdims=True)
    wts = e / (den + 1e-12)
    o_ref[...] = acc_ref[...] + lax.dot_general(
        wts, h, (((0,), (0,)), ((), ())), preferred_element_type=jnp.float32)


def _attpool(h, keep_col, batch_col, gw, gb2, acc):
    return pl.pallas_call(
        _attpool_body,
        in_specs=[pl.BlockSpec((N, C), lambda: (0, 0)),
                  pl.BlockSpec((N, 1), lambda: (0, 0)),
                  pl.BlockSpec((N, 1), lambda: (0, 0)),
                  pl.BlockSpec((C, 1), lambda: (0, 0)),
                  pl.BlockSpec((1, 1), lambda: (0, 0)),
                  pl.BlockSpec((G, C), lambda: (0, 0))],
        out_specs=pl.BlockSpec((G, C), lambda: (0, 0)),
        out_shape=jax.ShapeDtypeStruct((G, C), jnp.float32),
    )(h, keep_col, batch_col, gw, gb2, acc)


def _layer(h, keep_col, srcr4, dstr, zrows, w3, wih_t, whh_t, bih2, bhh2):
    m0, m1 = _m_step(h, keep_col, w3[0])
    for l in range(NGRU):
        a0, a1 = _edge_agg(m0, m1, srcr4, dstr, zrows)
        if l < NGRU - 1:
            h, m0, m1 = _gru(a0, a1, h, wih_t, whh_t, bih2, bhh2,
                             keep_col, w3[l + 1], final_relu=False)
        else:
            h = _gru(a0, a1, h, wih_t, whh_t, bih2, bhh2,
                     keep_col, None, final_relu=True)
    return h


def _topk(h, keep_col, batch_col, batch_row, keep_row, pw):
    score, wkey = _topk_score(h, keep_col, batch_col, pw.reshape(1, C))
    h, kn_col = _topk_rank(h, score, wkey, batch_col,
                           wkey.reshape(1, N), batch_row, keep_row)
    return h, kn_col


def kernel(x, edge_index, batch, lin_w, lin_b, ggc0_w, ggc0_wih, ggc0_whh,
           ggc0_bih, ggc0_bhh, pool0_w, ggc1_w, ggc1_wih, ggc1_whh,
           ggc1_bih, ggc1_bhh, pool1_w, gate_w, gate_b):
    src = edge_index[0].astype(jnp.int32)
    dst = edge_index[1].astype(jnp.int32)
    srcr = src.reshape(NS, NCHUNK, K)
    srcr4 = srcr[None] + (jnp.arange(4, dtype=jnp.int32) * N)[:, None, None, None]
    dstr = dst.reshape(NS, NCHUNK, K)
    zrows = jnp.zeros((RPS, 64), jnp.float32)
    bcol = batch.astype(jnp.float32).reshape(N, 1)
    brow = batch.astype(jnp.float32).reshape(1, N)
    ones_col = jnp.ones((N, 1), jnp.float32)
    ones_row = jnp.ones((1, N), jnp.float32)

    h = _lin(x, lin_w, lin_b.reshape(1, C))
    h = _layer(h, ones_col, srcr4, dstr, zrows, ggc0_w,
               ggc0_wih.T, ggc0_whh.T,
               ggc0_bih.reshape(1, 3 * C), ggc0_bhh.reshape(1, 3 * C))
    h, keep_col = _topk(h, ones_col, bcol, brow, ones_row, pool0_w)
    out = _attpool(h, keep_col, bcol, gate_w, gate_b.reshape(1, 1),
                   jnp.zeros((G, C), jnp.float32))
    h = _layer(h, keep_col, srcr4, dstr, zrows, ggc1_w,
               ggc1_wih.T, ggc1_whh.T,
               ggc1_bih.reshape(1, 3 * C), ggc1_bhh.reshape(1, 3 * C))
    h, keep_col = _topk(h, keep_col, bcol, brow,
                        keep_col.reshape(1, N), pool1_w)
    out = _attpool(h, keep_col, bcol, gate_w, gate_b.reshape(1, 1), out)
    return out
